# baseline (speedup 1.0000x reference)
import jax
import jax.numpy as jnp
from jax import lax
from jax.experimental import pallas as pl
from jax.experimental.pallas import tpu as pltpu
from jax.experimental.pallas import tpu_sc as plsc

_B = 8
_N = 100000
_TOTAL = _B * _N
_K = int(0.8 * _TOTAL)

_NTILES = 32
_CHUNK = _TOTAL // _NTILES
_CHUNK_PAD = 25024
_VECS = _CHUNK_PAD // 16
_NBUCKETS = 1 << 15


_ROWS = _TOTAL // 128
_CROWS = _ROWS // 10


def _tc1_body(x_ref, t_ref, w_ref, bits_ref, dicel_ref):
    col = lax.broadcasted_iota(jnp.int32, (1, 128), 1)
    inter = jnp.zeros((1, 128), jnp.float32)
    sum_p = jnp.zeros((1, 128), jnp.float32)
    sum_t = jnp.zeros((1, 128), jnp.float32)
    for i in range(_ROWS // _CROWS):
        r0 = i * _CROWS
        x = x_ref[r0 : r0 + _CROWS, :]
        t = t_ref[r0 : r0 + _CROWS, :]
        w = w_ref[r0 : r0 + _CROWS, :]
        e = jnp.exp(-jnp.abs(x))
        bce = (jnp.maximum(x, 0.0) - x * t + jnp.log(1.0 + e)) * w
        bits_ref[r0 : r0 + _CROWS, :] = lax.bitcast_convert_type(bce, jnp.int32)
        probs = 1.0 / (1.0 + jnp.exp(-x))
        pt = probs * t
        gid = (
            jnp.int32(r0 * 128)
            + lax.broadcasted_iota(jnp.int32, (_CROWS, 128), 0) * 128
            + lax.broadcasted_iota(jnp.int32, (_CROWS, 128), 1)
        )
        lo = (r0 * 128) // _N
        hi = (r0 * 128 + _CROWS * 128 - 1) // _N
        for b in range(lo, hi + 1):
            m = (gid >= b * _N) & (gid < (b + 1) * _N)
            lane = col == b
            inter += jnp.where(lane, jnp.sum(jnp.where(m, pt, 0.0)), 0.0)
            sum_p += jnp.where(lane, jnp.sum(jnp.where(m, probs, 0.0)), 0.0)
            sum_t += jnp.where(lane, jnp.sum(jnp.where(m, t, 0.0)), 0.0)
    dice_score = (2.0 * inter + 1e-06) / (sum_p + sum_t + 1e-06)
    z = jnp.abs(1.0 - dice_score)
    lc = z + jnp.log(1.0 + jnp.exp(-2.0 * z)) - jnp.float32(0.6931471805599453)
    dicel_ref[0, 0] = jnp.sum(jnp.where(col < _B, lc, 0.0)) / jnp.float32(_B)


def _sc_hist_body(bits_hbm, hist_hbm, data_v, hist_v):
    c = lax.axis_index("c")
    s = lax.axis_index("s")
    wid = c * 16 + s

    @plsc.parallel_loop(0, _NBUCKETS // 16, unroll=8)
    def _zero_body(j):
        hist_v[pl.ds(j * 16, 16)] = jnp.zeros((16,), jnp.int32)

    pltpu.sync_copy(
        bits_hbm.at[pl.ds(wid * _CHUNK, _CHUNK)], data_v.at[pl.ds(0, _CHUNK)]
    )
    data_v[pl.ds(_CHUNK, 16)] = jnp.zeros((16,), jnp.int32)
    data_v[pl.ds(_CHUNK + 8, 16)] = jnp.zeros((16,), jnp.int32)

    @plsc.parallel_loop(0, _VECS, unroll=4)
    def _hist_body(i):
        v = data_v[pl.ds(i * 16, 16)]
        idx = lax.shift_right_logical(v, 16)
        cnts, last = plsc.scan_count(idx)
        plsc.addupdate_scatter(hist_v, [idx], cnts, mask=last)

    pltpu.sync_copy(hist_v, hist_hbm.at[wid])


def _tc2_body(bits_ref, hists_ref, dicel_ref, out_ref):
    hist = hists_ref[0]
    for i in range(1, _NTILES):
        hist = hist + hists_ref[i]
    histf = hist.astype(jnp.float32)
    r = lax.broadcasted_iota(jnp.int32, (_NBUCKETS // 128, 128), 0)
    cc = lax.broadcasted_iota(jnp.int32, (_NBUCKETS // 128, 128), 1)
    bidx = r * 128 + cc
    kf = jnp.float32(_K)

    def bucket_search(i, tb):
        cand = tb | (jnp.int32(1) << (jnp.int32(14) - i))
        cnt = jnp.sum(jnp.where(bidx >= cand, histf, 0.0))
        return jnp.where(cnt >= kf, cand, tb)

    tbucket = lax.fori_loop(0, 15, bucket_search, jnp.int32(0))

    def low_search(i, tbits):
        cand = tbits | (jnp.int32(1) << (jnp.int32(15) - i))
        cnt = jnp.sum(jnp.where(bits_ref[:] >= cand, 1.0, 0.0))
        return jnp.where(cnt >= kf, cand, tbits)

    tbits = lax.fori_loop(0, 16, low_search, tbucket << 16)

    bits = bits_ref[:]
    vals = lax.bitcast_convert_type(bits, jnp.float32)
    gt = bits > tbits
    s_gt = jnp.sum(jnp.where(gt, vals, 0.0))
    c_gt = jnp.sum(jnp.where(gt, 1.0, 0.0))

    tvec = jnp.full((1, 128), tbits, jnp.int32)
    t_val = jnp.max(lax.bitcast_convert_type(tvec, jnp.float32))
    bce_loss = (s_gt + (kf - c_gt) * t_val) / kf
    dice_loss = dicel_ref[0, 0]
    out_ref[0, 0] = 1.0 * bce_loss + 10.0 * dice_loss
    out_ref[0, 1] = bce_loss
    out_ref[0, 2] = dice_loss


def kernel(pred_logits, target_labels, weights):
    x = pred_logits.reshape(_ROWS, 128)
    t = target_labels.reshape(_ROWS, 128)
    w = weights.reshape(_ROWS, 128)

    bits, dicel = pl.pallas_call(
        _tc1_body,
        out_shape=(
            jax.ShapeDtypeStruct((_ROWS, 128), jnp.int32),
            jax.ShapeDtypeStruct((1, 1), jnp.float32),
        ),
        out_specs=(
            pl.BlockSpec(memory_space=pltpu.VMEM),
            pl.BlockSpec(memory_space=pltpu.SMEM),
        ),
    )(x, t, w)

    sc_hist = pl.kernel(
        _sc_hist_body,
        out_type=jax.ShapeDtypeStruct((_NTILES, _NBUCKETS), jnp.int32),
        mesh=plsc.VectorSubcoreMesh(
            core_axis_name="c", subcore_axis_name="s", num_cores=2, num_subcores=16
        ),
        scratch_types=[
            pltpu.VMEM((_CHUNK_PAD,), jnp.int32),
            pltpu.VMEM((_NBUCKETS,), jnp.int32),
        ],
        compiler_params=pltpu.CompilerParams(needs_layout_passes=False),
    )
    hists = sc_hist(bits.reshape(_TOTAL))

    out = pl.pallas_call(
        _tc2_body,
        out_shape=jax.ShapeDtypeStruct((1, 3), jnp.float32),
        out_specs=pl.BlockSpec(memory_space=pltpu.SMEM),
        in_specs=(
            pl.BlockSpec(memory_space=pltpu.VMEM),
            pl.BlockSpec(memory_space=pltpu.VMEM),
            pl.BlockSpec(memory_space=pltpu.SMEM),
        ),
    )(bits, hists.reshape(_NTILES, _NBUCKETS // 128, 128), dicel)

    total = out[0, 0]
    bce_loss = out[0, 1]
    dice_loss = out[0, 2]
    return (total, lax.stop_gradient(bce_loss), lax.stop_gradient(dice_loss))


# speedup vs baseline: 1.0204x; 1.0204x over previous
import jax
import jax.numpy as jnp
from jax import lax
from jax.experimental import pallas as pl
from jax.experimental.pallas import tpu as pltpu
from jax.experimental.pallas import tpu_sc as plsc

_B = 8
_N = 100000
_TOTAL = _B * _N
_K = int(0.8 * _TOTAL)

_NTILES = 32
_CHUNK = _TOTAL // _NTILES
_CHUNK_PAD = 25024
_VECS = _CHUNK_PAD // 16
_NBUCKETS = 1 << 15


_ROWS = _TOTAL // 128
_CROWS = _ROWS // 10


def _tc1_body(x_ref, t_ref, w_ref, bits_ref, dicel_ref):
    col = lax.broadcasted_iota(jnp.int32, (1, 128), 1)
    inter = jnp.zeros((1, 128), jnp.float32)
    sum_p = jnp.zeros((1, 128), jnp.float32)
    sum_t = jnp.zeros((1, 128), jnp.float32)
    for i in range(_ROWS // _CROWS):
        r0 = i * _CROWS
        x = x_ref[r0 : r0 + _CROWS, :]
        t = t_ref[r0 : r0 + _CROWS, :]
        w = w_ref[r0 : r0 + _CROWS, :]
        e = jnp.exp(-jnp.abs(x))
        bce = (jnp.maximum(x, 0.0) - x * t + jnp.log(1.0 + e)) * w
        bits_ref[r0 : r0 + _CROWS, :] = lax.bitcast_convert_type(bce, jnp.int32)
        probs = 1.0 / (1.0 + jnp.exp(-x))
        pt = probs * t
        gid = (
            jnp.int32(r0 * 128)
            + lax.broadcasted_iota(jnp.int32, (_CROWS, 128), 0) * 128
            + lax.broadcasted_iota(jnp.int32, (_CROWS, 128), 1)
        )
        lo = (r0 * 128) // _N
        hi = (r0 * 128 + _CROWS * 128 - 1) // _N
        for b in range(lo, hi + 1):
            m = (gid >= b * _N) & (gid < (b + 1) * _N)
            lane = col == b
            inter += jnp.where(lane, jnp.sum(jnp.where(m, pt, 0.0)), 0.0)
            sum_p += jnp.where(lane, jnp.sum(jnp.where(m, probs, 0.0)), 0.0)
            sum_t += jnp.where(lane, jnp.sum(jnp.where(m, t, 0.0)), 0.0)
    dice_score = (2.0 * inter + 1e-06) / (sum_p + sum_t + 1e-06)
    z = jnp.abs(1.0 - dice_score)
    lc = z + jnp.log(1.0 + jnp.exp(-2.0 * z)) - jnp.float32(0.6931471805599453)
    dicel_ref[0, 0] = jnp.sum(jnp.where(col < _B, lc, 0.0)) / jnp.float32(_B)


def _sc_hist_body(bits_hbm, hist_hbm, data_v, hist_v):
    c = lax.axis_index("c")
    s = lax.axis_index("s")
    wid = c * 16 + s

    @plsc.parallel_loop(0, _NBUCKETS // 16, unroll=8)
    def _zero_body(j):
        hist_v[pl.ds(j * 16, 16)] = jnp.zeros((16,), jnp.int32)

    pltpu.sync_copy(
        bits_hbm.at[pl.ds(wid * _CHUNK, _CHUNK)], data_v.at[pl.ds(0, _CHUNK)]
    )
    data_v[pl.ds(_CHUNK, 16)] = jnp.zeros((16,), jnp.int32)
    data_v[pl.ds(_CHUNK + 8, 16)] = jnp.zeros((16,), jnp.int32)

    @plsc.parallel_loop(0, _VECS, unroll=4)
    def _hist_body(i):
        v = data_v[pl.ds(i * 16, 16)]
        idx = lax.shift_right_logical(v, 16)
        cnts, last = plsc.scan_count(idx)
        plsc.addupdate_scatter(hist_v, [idx], cnts, mask=last)

    pltpu.sync_copy(hist_v, hist_hbm.at[wid])


def _tc2_body(bits_ref, hists_ref, dicel_ref, out_ref, arr16_ref):
    hist = hists_ref[0]
    for i in range(1, _NTILES):
        hist = hist + hists_ref[i]
    histf = hist.astype(jnp.float32)
    r = lax.broadcasted_iota(jnp.int32, (_NBUCKETS // 128, 128), 0)
    cc = lax.broadcasted_iota(jnp.int32, (_NBUCKETS // 128, 128), 1)
    bidx = r * 128 + cc
    kf = jnp.float32(_K)

    def bucket_search(i, tb):
        cand = tb | (jnp.int32(1) << (jnp.int32(14) - i))
        cnt = jnp.sum(jnp.where(bidx >= cand, histf, 0.0))
        return jnp.where(cnt >= kf, cand, tb)

    tbucket = lax.fori_loop(0, 15, bucket_search, jnp.int32(0))

    c_above = jnp.sum(jnp.where(bidx > tbucket, histf, 0.0))

    bits = bits_ref[:]
    vals = lax.bitcast_convert_type(bits, jnp.float32)
    bucket = lax.shift_right_logical(bits, 16)
    s_above = jnp.sum(jnp.where(bucket > tbucket, vals, 0.0))
    low = bits & jnp.int32(0xFFFF)
    arr16_ref[:] = jnp.where(
        bucket == tbucket, low - 32768, jnp.int32(-32768)
    ).astype(jnp.int16)

    def low_search(i, lowt):
        cand = lowt | (jnp.int32(1) << (jnp.int32(15) - i))
        aw = arr16_ref[:].astype(jnp.int32)
        cnt = c_above + jnp.sum(jnp.where(aw >= cand - 32768, 1.0, 0.0))
        return jnp.where(cnt >= kf, cand, lowt)

    lowt = lax.fori_loop(0, 16, low_search, jnp.int32(0))

    a32 = arr16_ref[:].astype(jnp.int32)
    gt = a32 > (lowt - 32768)
    lowv = (a32 & jnp.int32(0xFFFF)) ^ jnp.int32(0x8000)
    v_b = lax.bitcast_convert_type((tbucket << 16) | lowv, jnp.float32)
    s_gt = s_above + jnp.sum(jnp.where(gt, v_b, 0.0))
    c_gt = c_above + jnp.sum(jnp.where(gt, 1.0, 0.0))

    tbits = (tbucket << 16) | lowt
    tvec = jnp.full((1, 128), tbits, jnp.int32)
    t_val = jnp.max(lax.bitcast_convert_type(tvec, jnp.float32))
    bce_loss = (s_gt + (kf - c_gt) * t_val) / kf
    dice_loss = dicel_ref[0, 0]
    out_ref[0, 0] = 1.0 * bce_loss + 10.0 * dice_loss
    out_ref[0, 1] = bce_loss
    out_ref[0, 2] = dice_loss


def kernel(pred_logits, target_labels, weights):
    x = pred_logits.reshape(_ROWS, 128)
    t = target_labels.reshape(_ROWS, 128)
    w = weights.reshape(_ROWS, 128)

    bits, dicel = pl.pallas_call(
        _tc1_body,
        out_shape=(
            jax.ShapeDtypeStruct((_ROWS, 128), jnp.int32),
            jax.ShapeDtypeStruct((1, 1), jnp.float32),
        ),
        out_specs=(
            pl.BlockSpec(memory_space=pltpu.VMEM),
            pl.BlockSpec(memory_space=pltpu.SMEM),
        ),
    )(x, t, w)

    sc_hist = pl.kernel(
        _sc_hist_body,
        out_type=jax.ShapeDtypeStruct((_NTILES, _NBUCKETS), jnp.int32),
        mesh=plsc.VectorSubcoreMesh(
            core_axis_name="c", subcore_axis_name="s", num_cores=2, num_subcores=16
        ),
        scratch_types=[
            pltpu.VMEM((_CHUNK_PAD,), jnp.int32),
            pltpu.VMEM((_NBUCKETS,), jnp.int32),
        ],
        compiler_params=pltpu.CompilerParams(needs_layout_passes=False),
    )
    hists = sc_hist(bits.reshape(_TOTAL))

    out = pl.pallas_call(
        _tc2_body,
        out_shape=jax.ShapeDtypeStruct((1, 3), jnp.float32),
        out_specs=pl.BlockSpec(memory_space=pltpu.SMEM),
        in_specs=(
            pl.BlockSpec(memory_space=pltpu.VMEM),
            pl.BlockSpec(memory_space=pltpu.VMEM),
            pl.BlockSpec(memory_space=pltpu.SMEM),
        ),
        scratch_shapes=[pltpu.VMEM((_ROWS, 128), jnp.int16)],
    )(bits, hists.reshape(_NTILES, _NBUCKETS // 128, 128), dicel)

    total = out[0, 0]
    bce_loss = out[0, 1]
    dice_loss = out[0, 2]
    return (total, lax.stop_gradient(bce_loss), lax.stop_gradient(dice_loss))


# speedup vs baseline: 1.4276x; 1.3991x over previous
import jax
import jax.numpy as jnp
from jax import lax
from jax.experimental import pallas as pl
from jax.experimental.pallas import tpu as pltpu
from jax.experimental.pallas import tpu_sc as plsc

_B = 8
_N = 100000
_NPAD = 100096
_TOTPAD = _B * _NPAD
_K = int(0.8 * _B * _N)

_NTILES = 32
_CHUNK = _TOTPAD // _NTILES
_VECS = _CHUNK // 16
_NBUCKETS = 1 << 15


def _tc1_body(x_ref, t_ref, w_ref, bits_ref, dicel_ref):
    x = x_ref[:]
    t = t_ref[:]
    w = w_ref[:]
    e = jnp.exp(-jnp.abs(x))
    bce = (jnp.maximum(x, 0.0) - x * t + jnp.log(1.0 + e)) * w
    bits_ref[:, pl.ds(_NPAD - 128, 128)] = jnp.zeros((_B, 128), jnp.int32)
    bits_ref[:, pl.ds(0, _N)] = lax.bitcast_convert_type(bce, jnp.int32)
    probs = 1.0 / (1.0 + jnp.exp(-x))
    inter = jnp.sum(probs * t, axis=1, keepdims=True)
    sum_p = jnp.sum(probs, axis=1, keepdims=True)
    sum_t = jnp.sum(t, axis=1, keepdims=True)
    dice_score = (2.0 * inter + 1e-06) / (sum_p + sum_t + 1e-06)
    z = jnp.abs(1.0 - dice_score)
    lc = z + jnp.log(1.0 + jnp.exp(-2.0 * z)) - jnp.float32(0.6931471805599453)
    dicel_ref[0, 0] = jnp.sum(lc) / jnp.float32(_B)


def _sc_hist_body(bits_hbm, hist_hbm, data_v, hist_v):
    c = lax.axis_index("c")
    s = lax.axis_index("s")
    wid = c * 16 + s

    @plsc.parallel_loop(0, _NBUCKETS // 16, unroll=8)
    def _zero_body(j):
        hist_v[pl.ds(j * 16, 16)] = jnp.zeros((16,), jnp.int32)

    pltpu.sync_copy(bits_hbm.at[pl.ds(wid * _CHUNK, _CHUNK)], data_v)

    @plsc.parallel_loop(0, _VECS, unroll=4)
    def _hist_body(i):
        v = data_v[pl.ds(i * 16, 16)]
        idx = lax.shift_right_logical(v, 16)
        cnts, last = plsc.scan_count(idx)
        plsc.addupdate_scatter(hist_v, [idx], cnts, mask=last)

    pltpu.sync_copy(hist_v, hist_hbm.at[wid])


def _tc2_body(bits_ref, hists_ref, dicel_ref, out_ref, arr16_ref):
    hist = jnp.sum(hists_ref[:], axis=0, keepdims=True)
    histf = hist.astype(jnp.float32)
    bidx = lax.broadcasted_iota(jnp.int32, (1, _NBUCKETS), 1)
    kf = jnp.float32(_K)

    def bucket_search(i, tb):
        cand = tb | (jnp.int32(1) << (jnp.int32(14) - i))
        cnt = jnp.sum(jnp.where(bidx >= cand, histf, 0.0))
        return jnp.where(cnt >= kf, cand, tb)

    tbucket = lax.fori_loop(0, 15, bucket_search, jnp.int32(0))

    c_above = jnp.sum(jnp.where(bidx > tbucket, histf, 0.0))

    bits = bits_ref[:]
    vals = lax.bitcast_convert_type(bits, jnp.float32)
    bucket = lax.shift_right_logical(bits, 16)
    s_above = jnp.sum(jnp.where(bucket > tbucket, vals, 0.0))
    low = bits & jnp.int32(0xFFFF)
    arr16_ref[:] = jnp.where(
        bucket == tbucket, low - 32768, jnp.int32(-32768)
    ).astype(jnp.int16)

    def low_search(i, lowt):
        cand = lowt | (jnp.int32(1) << (jnp.int32(15) - i))
        aw = arr16_ref[:].astype(jnp.int32)
        cnt = c_above + jnp.sum(jnp.where(aw >= cand - 32768, 1.0, 0.0))
        return jnp.where(cnt >= kf, cand, lowt)

    lowt = lax.fori_loop(0, 16, low_search, jnp.int32(0))

    a32 = arr16_ref[:].astype(jnp.int32)
    gt = a32 > (lowt - 32768)
    lowv = (a32 & jnp.int32(0xFFFF)) ^ jnp.int32(0x8000)
    v_b = lax.bitcast_convert_type((tbucket << 16) | lowv, jnp.float32)
    s_gt = s_above + jnp.sum(jnp.where(gt, v_b, 0.0))
    c_gt = c_above + jnp.sum(jnp.where(gt, 1.0, 0.0))

    tbits = (tbucket << 16) | lowt
    tvec = jnp.full((1, 128), tbits, jnp.int32)
    t_val = jnp.max(lax.bitcast_convert_type(tvec, jnp.float32))
    bce_loss = (s_gt + (kf - c_gt) * t_val) / kf
    dice_loss = dicel_ref[0, 0]
    out_ref[0, 0] = 1.0 * bce_loss + 10.0 * dice_loss
    out_ref[0, 1] = bce_loss
    out_ref[0, 2] = dice_loss


def kernel(pred_logits, target_labels, weights):
    x = pred_logits.reshape(_B, _N)
    t = target_labels.reshape(_B, _N)
    w = weights.reshape(_B, _N)

    bits, dicel = pl.pallas_call(
        _tc1_body,
        out_shape=(
            jax.ShapeDtypeStruct((_B, _NPAD), jnp.int32),
            jax.ShapeDtypeStruct((1, 1), jnp.float32),
        ),
        out_specs=(
            pl.BlockSpec(memory_space=pltpu.VMEM),
            pl.BlockSpec(memory_space=pltpu.SMEM),
        ),
    )(x, t, w)

    sc_hist = pl.kernel(
        _sc_hist_body,
        out_type=jax.ShapeDtypeStruct((_NTILES, _NBUCKETS), jnp.int32),
        mesh=plsc.VectorSubcoreMesh(
            core_axis_name="c", subcore_axis_name="s", num_cores=2, num_subcores=16
        ),
        scratch_types=[
            pltpu.VMEM((_CHUNK,), jnp.int32),
            pltpu.VMEM((_NBUCKETS,), jnp.int32),
        ],
        compiler_params=pltpu.CompilerParams(needs_layout_passes=False),
    )
    hists = sc_hist(bits.reshape(_TOTPAD))

    out = pl.pallas_call(
        _tc2_body,
        out_shape=jax.ShapeDtypeStruct((1, 3), jnp.float32),
        out_specs=pl.BlockSpec(memory_space=pltpu.SMEM),
        in_specs=(
            pl.BlockSpec(memory_space=pltpu.VMEM),
            pl.BlockSpec(memory_space=pltpu.VMEM),
            pl.BlockSpec(memory_space=pltpu.SMEM),
        ),
        scratch_shapes=[pltpu.VMEM((_B, _NPAD), jnp.int16)],
    )(bits, hists, dicel)

    total = out[0, 0]
    bce_loss = out[0, 1]
    dice_loss = out[0, 2]
    return (total, lax.stop_gradient(bce_loss), lax.stop_gradient(dice_loss))


# speedup vs baseline: 1.5356x; 1.0757x over previous
import jax
import jax.numpy as jnp
from jax import lax
from jax.experimental import pallas as pl
from jax.experimental.pallas import tpu as pltpu
from jax.experimental.pallas import tpu_sc as plsc

_B = 8
_N = 100000
_NPAD = 100096
_TOTPAD = _B * _NPAD
_K = int(0.8 * _B * _N)

_NTILES = 32
_CHUNK = _TOTPAD // _NTILES
_VECS = _CHUNK // 16
_NBUCKETS = 1 << 15


def _tc1_body(x_ref, t_ref, w_ref, bits_ref, dicel_ref):
    x = x_ref[:]
    t = t_ref[:]
    w = w_ref[:]
    e = jnp.exp(-jnp.abs(x))
    bce = (jnp.maximum(x, 0.0) - x * t + jnp.log(1.0 + e)) * w
    bits_ref[:, pl.ds(_NPAD - 128, 128)] = jnp.zeros((_B, 128), jnp.int32)
    bits_ref[:, pl.ds(0, _N)] = lax.bitcast_convert_type(bce, jnp.int32)
    probs = 1.0 / (1.0 + jnp.exp(-x))
    inter = jnp.sum(probs * t, axis=1, keepdims=True)
    sum_p = jnp.sum(probs, axis=1, keepdims=True)
    sum_t = jnp.sum(t, axis=1, keepdims=True)
    dice_score = (2.0 * inter + 1e-06) / (sum_p + sum_t + 1e-06)
    z = jnp.abs(1.0 - dice_score)
    lc = z + jnp.log(1.0 + jnp.exp(-2.0 * z)) - jnp.float32(0.6931471805599453)
    dicel_ref[0, 0] = jnp.sum(lc) / jnp.float32(_B)


def _sc_hist_body(bits_hbm, hist_hbm, data_v, hist_v):
    c = lax.axis_index("c")
    s = lax.axis_index("s")
    wid = c * 16 + s

    @plsc.parallel_loop(0, _NBUCKETS // 16, unroll=8)
    def _zero_body(j):
        hist_v[pl.ds(j * 16, 16)] = jnp.zeros((16,), jnp.int32)

    pltpu.sync_copy(bits_hbm.at[pl.ds(wid * _CHUNK, _CHUNK)], data_v)

    @plsc.parallel_loop(0, _VECS, unroll=4)
    def _hist_body(i):
        v = data_v[pl.ds(i * 16, 16)]
        idx = lax.shift_right_logical(v, 16)
        cnts, last = plsc.scan_count(idx)
        plsc.addupdate_scatter(hist_v, [idx], cnts, mask=last)

    pltpu.sync_copy(hist_v, hist_hbm.at[wid])


def _tc2_body(bits_ref, hists_ref, dicel_ref, out_ref, arr16_ref):
    hist = jnp.sum(hists_ref[:], axis=0, keepdims=True)
    histf = hist.astype(jnp.float32)
    bidx = lax.broadcasted_iota(jnp.int32, (1, _NBUCKETS), 1)
    kf = jnp.float32(_K)

    def bucket_round(i, tb):
        hi = jnp.int32(1) << (jnp.int32(14) - 2 * i)
        lo = jnp.int32(1) << (jnp.int32(13) - 2 * i)

        def cnt(c):
            return jnp.sum(jnp.where(bidx >= c, histf, 0.0))

        c11 = cnt(tb | hi | lo)
        c10 = cnt(tb | hi)
        c01 = cnt(tb | lo)
        return jnp.where(
            c11 >= kf,
            tb | hi | lo,
            jnp.where(c10 >= kf, tb | hi, jnp.where(c01 >= kf, tb | lo, tb)),
        )

    tbucket = lax.fori_loop(0, 7, bucket_round, jnp.int32(0))
    cand0 = tbucket | jnp.int32(1)
    cnt0 = jnp.sum(jnp.where(bidx >= cand0, histf, 0.0))
    tbucket = jnp.where(cnt0 >= kf, cand0, tbucket)

    c_above = jnp.sum(jnp.where(bidx > tbucket, histf, 0.0))

    bits = bits_ref[:]
    vals = lax.bitcast_convert_type(bits, jnp.float32)
    bucket = lax.shift_right_logical(bits, 16)
    s_above = jnp.sum(jnp.where(bucket > tbucket, vals, 0.0))
    low = bits & jnp.int32(0xFFFF)
    arr16_ref[:] = jnp.where(
        bucket == tbucket, low - 32768, jnp.int32(-32768)
    ).astype(jnp.int16)

    def low_round(i, lowt):
        hi = jnp.int32(1) << (jnp.int32(15) - 2 * i)
        lo = jnp.int32(1) << (jnp.int32(14) - 2 * i)
        aw = arr16_ref[:].astype(jnp.int32)

        def cnt(c):
            return c_above + jnp.sum(jnp.where(aw >= c - 32768, 1.0, 0.0))

        c11 = cnt(lowt | hi | lo)
        c10 = cnt(lowt | hi)
        c01 = cnt(lowt | lo)
        return jnp.where(
            c11 >= kf,
            lowt | hi | lo,
            jnp.where(
                c10 >= kf, lowt | hi, jnp.where(c01 >= kf, lowt | lo, lowt)
            ),
        )

    lowt = lax.fori_loop(0, 8, low_round, jnp.int32(0))

    a32 = arr16_ref[:].astype(jnp.int32)
    gt = a32 > (lowt - 32768)
    lowv = (a32 & jnp.int32(0xFFFF)) ^ jnp.int32(0x8000)
    v_b = lax.bitcast_convert_type((tbucket << 16) | lowv, jnp.float32)
    s_gt = s_above + jnp.sum(jnp.where(gt, v_b, 0.0))
    c_gt = c_above + jnp.sum(jnp.where(gt, 1.0, 0.0))

    tbits = (tbucket << 16) | lowt
    tvec = jnp.full((1, 128), tbits, jnp.int32)
    t_val = jnp.max(lax.bitcast_convert_type(tvec, jnp.float32))
    bce_loss = (s_gt + (kf - c_gt) * t_val) / kf
    dice_loss = dicel_ref[0, 0]
    out_ref[0, 0] = 1.0 * bce_loss + 10.0 * dice_loss
    out_ref[0, 1] = bce_loss
    out_ref[0, 2] = dice_loss


def kernel(pred_logits, target_labels, weights):
    x = pred_logits.reshape(_B, _N)
    t = target_labels.reshape(_B, _N)
    w = weights.reshape(_B, _N)

    bits, dicel = pl.pallas_call(
        _tc1_body,
        out_shape=(
            jax.ShapeDtypeStruct((_B, _NPAD), jnp.int32),
            jax.ShapeDtypeStruct((1, 1), jnp.float32),
        ),
        out_specs=(
            pl.BlockSpec(memory_space=pltpu.VMEM),
            pl.BlockSpec(memory_space=pltpu.SMEM),
        ),
    )(x, t, w)

    sc_hist = pl.kernel(
        _sc_hist_body,
        out_type=jax.ShapeDtypeStruct((_NTILES, _NBUCKETS), jnp.int32),
        mesh=plsc.VectorSubcoreMesh(
            core_axis_name="c", subcore_axis_name="s", num_cores=2, num_subcores=16
        ),
        scratch_types=[
            pltpu.VMEM((_CHUNK,), jnp.int32),
            pltpu.VMEM((_NBUCKETS,), jnp.int32),
        ],
        compiler_params=pltpu.CompilerParams(needs_layout_passes=False),
    )
    hists = sc_hist(bits.reshape(_TOTPAD))

    out = pl.pallas_call(
        _tc2_body,
        out_shape=jax.ShapeDtypeStruct((1, 3), jnp.float32),
        out_specs=pl.BlockSpec(memory_space=pltpu.SMEM),
        in_specs=(
            pl.BlockSpec(memory_space=pltpu.VMEM),
            pl.BlockSpec(memory_space=pltpu.VMEM),
            pl.BlockSpec(memory_space=pltpu.SMEM),
        ),
        scratch_shapes=[pltpu.VMEM((_B, _NPAD), jnp.int16)],
    )(bits, hists, dicel)

    total = out[0, 0]
    bce_loss = out[0, 1]
    dice_loss = out[0, 2]
    return (total, lax.stop_gradient(bce_loss), lax.stop_gradient(dice_loss))
